# Initial kernel scaffold; baseline (speedup 1.0000x reference)
import jax
import jax.numpy as jnp
from jax.experimental import pallas as pl


def kernel(tokens, embedding):
    raise NotImplementedError("write your pallas kernel here")


# speedup vs baseline: 4.1121x; 4.1121x over previous
import functools
import jax
import jax.numpy as jnp
from jax import lax
from jax.experimental import pallas as pl
from jax.experimental.pallas import tpu as pltpu
from jax.experimental.pallas import tpu_sc as plsc

V = 100000
D = 64
B = 200 * 4096
NC, NS = 2, 16
NW = NC * NS
BPW = B // NW
CH = 128
NCHUNK = BPW // CH
NB = 8
KA = 4

_SCALE = 8.0


def _scale_body(x_ref, o_ref):
    o_ref[...] = x_ref[...] * _SCALE


def _scale_table(emb):
    rows = 5000
    return pl.pallas_call(
        _scale_body,
        out_shape=jax.ShapeDtypeStruct((V, D), jnp.float32),
        grid=(V // rows,),
        in_specs=[pl.BlockSpec((rows, D), lambda i: (i, 0))],
        out_specs=pl.BlockSpec((rows, D), lambda i: (i, 0)),
    )(emb)


def _gather_body(table, idxs, out, idx_v, rows_v, gsem, osem):
    wid = lax.axis_index("s") * NC + lax.axis_index("c")
    base = wid * BPW
    pltpu.sync_copy(idxs.at[wid], idx_v)

    def _issue_gather(j, b):
        pltpu.async_copy(table.at[idx_v.at[j]], rows_v.at[b], gsem.at[b])

    def _wait_gather(j, b):
        pltpu.make_async_copy(table.at[idx_v.at[j]], rows_v.at[b],
                              gsem.at[b]).wait()

    def _issue_out(j, b):
        pltpu.async_copy(rows_v.at[b], out.at[pl.ds(base + j * CH, CH)],
                         osem.at[b])

    def _wait_out(j, b):
        pltpu.make_async_copy(rows_v.at[b], out.at[pl.ds(base + j * CH, CH)],
                              osem.at[b]).wait()

    for j in range(KA):
        _issue_gather(j, j % NB)

    def outer(g, carry):
        for b in range(NB):
            j = g * NB + b
            _wait_gather(j, b)
            _issue_out(j, b)
            jn = j + KA
            bn = (b + KA) % NB

            @pl.when(jn < NCHUNK)
            def _():
                @pl.when(jn >= NB)
                def _():
                    _wait_out(jn - NB, bn)
                _issue_gather(jn, bn)
        return carry

    lax.fori_loop(0, NCHUNK // NB, outer, 0)

    for i in range(KA):
        j = NCHUNK - KA + i
        _wait_out(j, j % NB)


def _gather(table, idx3):
    mesh = plsc.VectorSubcoreMesh(core_axis_name="c", subcore_axis_name="s")
    f = pl.kernel(
        _gather_body,
        mesh=mesh,
        compiler_params=pltpu.CompilerParams(use_tc_tiling_on_sc=False),
        out_type=jax.ShapeDtypeStruct((B, D), jnp.float32),
        scratch_types=[
            pltpu.VMEM((NCHUNK, CH), jnp.int32),
            pltpu.VMEM((NB, CH, D), jnp.float32),
            pltpu.SemaphoreType.DMA((NB,)),
            pltpu.SemaphoreType.DMA((NB,)),
        ],
    )
    return f(table, idx3)


def kernel(tokens, embedding):
    t = tokens.reshape(NW, NCHUNK, CH).astype(jnp.int32)
    scaled = _scale_table(embedding)
    flat = _gather(scaled, t)
    return flat.reshape(tokens.shape[0], tokens.shape[1], D)
